# baseline (speedup 1.0000x reference)
import functools

import jax
import jax.numpy as jnp
from jax import lax
from jax.experimental import pallas as pl
from jax.experimental.pallas import tpu as pltpu
from jax.experimental.pallas import tpu_sc as plsc

GRP = 8


def _emb_kernel(b, s, d, per_w):
    mesh = plsc.VectorSubcoreMesh(core_axis_name="c", subcore_axis_name="s")
    info = plsc.get_sparse_core_info()
    nc = info.num_cores
    n_groups = per_w // GRP
    s_pad = (s + 7) // 8 * 8
    d_pad = (d + 127) // 128 * 128

    @functools.partial(
        pl.kernel,
        mesh=mesh,
        compiler_params=pltpu.CompilerParams(use_tc_tiling_on_sc=False),
        out_type=jax.ShapeDtypeStruct((b, s_pad, d_pad), jnp.float32),
        scratch_types=[
            pltpu.VMEM((2, GRP, s), jnp.int32),
            pltpu.VMEM((2, GRP, s, d), jnp.float32),
            pltpu.SemaphoreType.DMA,
            pltpu.SemaphoreType.DMA,
            pltpu.SemaphoreType.DMA,
        ],
    )
    def emb(ids_hbm, table_hbm, out_hbm, idx_v, rows_v, isem, gsem, wsem):
        wid = lax.axis_index("s") * nc + lax.axis_index("c")
        tok0 = wid * per_w

        def stage_idx(g, bank):
            pltpu.make_async_copy(
                ids_hbm.at[pl.ds(tok0 + g * GRP, GRP)], idx_v.at[bank], isem
            ).start()

        def wait_idx(bank):
            pltpu.make_async_copy(
                ids_hbm.at[pl.ds(tok0, GRP)], idx_v.at[bank], isem
            ).wait()

        def fire_gathers(bank):
            for t in range(GRP):
                pltpu.make_async_copy(
                    table_hbm.at[idx_v.at[bank, t]], rows_v.at[bank, t], gsem
                ).start()

        def drain_gathers(bank):
            for t in range(GRP):
                pltpu.make_async_copy(
                    table_hbm.at[idx_v.at[bank, t]], rows_v.at[bank, t], gsem
                ).wait()

        stage_idx(0, 0)
        stage_idx(1, 1)
        wait_idx(0)
        fire_gathers(0)

        def body(g, _):
            bank = lax.rem(g, 2)

            @pl.when(g > 0)
            def _():
                pltpu.make_async_copy(
                    rows_v.at[1 - bank],
                    out_hbm.at[pl.ds(tok0, GRP), pl.ds(0, s), pl.ds(0, d)],
                    wsem,
                ).wait()

            @pl.when(g + 1 < n_groups)
            def _():
                wait_idx(1 - bank)
                fire_gathers(1 - bank)

            drain_gathers(bank)

            @pl.when(g + 2 < n_groups)
            def _():
                stage_idx(g + 2, bank)

            pltpu.make_async_copy(
                rows_v.at[bank],
                out_hbm.at[pl.ds(tok0 + g * GRP, GRP), pl.ds(0, s), pl.ds(0, d)],
                wsem,
            ).start()
            return 0

        lax.fori_loop(0, n_groups, body, 0)

        pltpu.make_async_copy(
            rows_v.at[(n_groups - 1) % 2],
            out_hbm.at[pl.ds(tok0, GRP), pl.ds(0, s), pl.ds(0, d)],
            wsem,
        ).wait()

    return emb


def _transpose_kernel(d, v):
    mesh = plsc.VectorSubcoreMesh(core_axis_name="c", subcore_axis_name="s")
    info = plsc.get_sparse_core_info()
    nc = info.num_cores
    nw = nc * info.num_subcores
    n_full = v // 128
    tail = v - n_full * 128
    n_tiles = n_full + (1 if tail else 0)
    tail_w = n_full % nw
    per_row = 128 // d

    @functools.partial(
        pl.kernel,
        mesh=mesh,
        compiler_params=pltpu.CompilerParams(
            use_tc_tiling_on_sc=True, needs_layout_passes=False
        ),
        out_type=jax.ShapeDtypeStruct((n_tiles * d, 128), jnp.float32),
        scratch_types=[
            pltpu.VMEM((2, d, 128), jnp.float32),
            pltpu.VMEM((2, d, 128), jnp.float32),
            pltpu.VMEM((d, tail or 128), jnp.float32),
            pltpu.SemaphoreType.DMA,
            pltpu.SemaphoreType.DMA,
        ],
    )
    def trans(tt_hbm, tailt_hbm, out_hbm, in_v, out_v, tail_v, isem, wsem):
        wid = lax.axis_index("s") * nc + lax.axis_index("c")
        n_mine = (n_tiles - 1 - wid) // nw + 1
        iotas = [lax.iota(jnp.int32, 16) + 16 * k for k in range(d // 16)]

        def start_in(k, bank):
            c = wid + k * nw

            @pl.when(c < n_full)
            def _():
                pltpu.make_async_copy(
                    tt_hbm.at[:, pl.ds(c * 128, 128)], in_v.at[bank], isem
                ).start()

        def wait_in(bank):
            pltpu.make_async_copy(
                tt_hbm.at[:, pl.ds(0, 128)], in_v.at[bank], isem
            ).wait()

        def transpose_rows(src_v, bank, n_rows):
            for r in range(n_rows):
                for h in range(per_row):
                    col = jnp.full((16,), per_row * r + h, jnp.int32)
                    for kk in range(d // 16):
                        vals = plsc.load_gather(src_v, [iotas[kk], col])
                        out_v[bank, r, pl.ds(h * d + 16 * kk, 16)] = vals

        def body(k, _):
            bank = lax.rem(k, 2)
            c = wid + k * nw

            @pl.when(k + 1 < n_mine)
            def _():
                start_in(k + 1, 1 - bank)

            @pl.when(k >= 2)
            def _():
                pltpu.make_async_copy(
                    out_v.at[bank], out_hbm.at[pl.ds(0, d)], wsem
                ).wait()

            @pl.when(c < n_full)
            def _():
                wait_in(bank)
                transpose_rows(in_v.at[bank], bank, d)
                pltpu.make_async_copy(
                    out_v.at[bank], out_hbm.at[pl.ds(c * d, d)], wsem
                ).start()

            if tail:
                @pl.when(c == n_full)
                def _():
                    pltpu.sync_copy(tailt_hbm, tail_v)
                    transpose_rows(tail_v, bank, d * tail // 128)
                    pltpu.make_async_copy(
                        out_v.at[bank, pl.ds(0, d * tail // 128)],
                        out_hbm.at[pl.ds(c * d, d * tail // 128)],
                        wsem,
                    ).start()

            return 0

        start_in(0, 0)
        lax.fori_loop(0, n_mine, body, 0)

        pltpu.make_async_copy(
            out_v.at[0], out_hbm.at[pl.ds(0, d)], wsem
        ).wait()
        if tail:
            @pl.when(wid == tail_w)
            def _():
                pltpu.make_async_copy(
                    out_v.at[0, pl.ds(0, d * tail // 128)],
                    out_hbm.at[pl.ds(0, d * tail // 128)],
                    wsem,
                ).wait()

            @pl.when(wid != tail_w)
            def _():
                pltpu.make_async_copy(
                    out_v.at[0], out_hbm.at[pl.ds(0, d)], wsem
                ).wait()
        else:
            pltpu.make_async_copy(
                out_v.at[0], out_hbm.at[pl.ds(0, d)], wsem
            ).wait()

    return trans


def kernel(token_ids, embdM):
    b, s = token_ids.shape
    v, d = embdM.shape
    info = plsc.get_sparse_core_info()
    nw = info.num_cores * info.num_subcores
    per_w = b // nw

    n_full = v // 128
    tableT = embdM.T
    tailT = tableT[:, n_full * 128 :]
    packed = _transpose_kernel(d, v)(tableT, tailT)
    table_c = packed.reshape(-1, d)
    out_p = _emb_kernel(b, s, d, per_w)(token_ids.astype(jnp.int32), table_c)
    return out_p[:, :s, :d]


# speedup vs baseline: 1.9572x; 1.9572x over previous
import functools

import jax
import jax.numpy as jnp
from jax import lax
from jax.experimental import pallas as pl
from jax.experimental.pallas import tpu as pltpu
from jax.experimental.pallas import tpu_sc as plsc

GRP = 8


def _emb_kernel(b, s, d, per_w):
    mesh = plsc.VectorSubcoreMesh(core_axis_name="c", subcore_axis_name="s")
    info = plsc.get_sparse_core_info()
    nc = info.num_cores
    n_groups = per_w // GRP
    s_pad = (s + 7) // 8 * 8
    d_pad = (d + 127) // 128 * 128

    @functools.partial(
        pl.kernel,
        mesh=mesh,
        compiler_params=pltpu.CompilerParams(use_tc_tiling_on_sc=False),
        out_type=jax.ShapeDtypeStruct((b, s_pad, d_pad), jnp.float32),
        scratch_types=[
            pltpu.VMEM((2, GRP, s), jnp.int32),
            pltpu.VMEM((2, GRP, s, d), jnp.float32),
            pltpu.SemaphoreType.DMA,
            pltpu.SemaphoreType.DMA,
            pltpu.SemaphoreType.DMA,
            pltpu.SemaphoreType.DMA,
            pltpu.SemaphoreType.DMA,
        ],
    )
    def emb(ids_hbm, table_hbm, out_hbm, idx_v, rows_v,
            isem0, isem1, gsem0, gsem1, wsem):
        wid = lax.axis_index("s") * nc + lax.axis_index("c")
        tok0 = wid * per_w
        isems = (isem0, isem1)
        gsems = (gsem0, gsem1)


        def stage_idx(g, bank):
            pltpu.make_async_copy(
                ids_hbm.at[pl.ds(tok0 + g * GRP, GRP)], idx_v.at[bank],
                isems[bank],
            ).start()

        def wait_idx(bank):
            pltpu.make_async_copy(
                ids_hbm.at[pl.ds(tok0, GRP)], idx_v.at[bank], isems[bank]
            ).wait()

        def fire_gathers(bank):
            for t in range(GRP):
                pltpu.make_async_copy(
                    table_hbm.at[idx_v.at[bank, t]], rows_v.at[bank, t],
                    gsems[bank],
                ).start()

        def drain_gathers(bank):
            for t in range(GRP):
                pltpu.make_async_copy(
                    table_hbm.at[idx_v.at[bank, t]], rows_v.at[bank, t],
                    gsems[bank],
                ).wait()

        stage_idx(0, 0)
        stage_idx(1, 1)
        wait_idx(0)
        fire_gathers(0)

        def half(g, bank):
            @pl.when(g > 0)
            def _():
                pltpu.make_async_copy(
                    rows_v.at[1 - bank],
                    out_hbm.at[pl.ds(tok0, GRP), pl.ds(0, s), pl.ds(0, d)],
                    wsem,
                ).wait()

            @pl.when(g + 1 < n_groups)
            def _():
                wait_idx(1 - bank)
                fire_gathers(1 - bank)

            drain_gathers(bank)

            @pl.when(g + 2 < n_groups)
            def _():
                stage_idx(g + 2, bank)

            pltpu.make_async_copy(
                rows_v.at[bank],
                out_hbm.at[pl.ds(tok0 + g * GRP, GRP), pl.ds(0, s), pl.ds(0, d)],
                wsem,
            ).start()

        def body(g, _):
            is_even = lax.rem(g, 2) == 0

            @pl.when(is_even)
            def _():
                half(g, 0)

            @pl.when(jnp.logical_not(is_even))
            def _():
                half(g, 1)

            return 0

        lax.fori_loop(0, n_groups, body, 0)

        pltpu.make_async_copy(
            rows_v.at[(n_groups - 1) % 2],
            out_hbm.at[pl.ds(tok0, GRP), pl.ds(0, s), pl.ds(0, d)],
            wsem,
        ).wait()

    return emb


def kernel(token_ids, embdM):
    b, s = token_ids.shape
    d = embdM.shape[1]
    info = plsc.get_sparse_core_info()
    nw = info.num_cores * info.num_subcores
    per_w = b // nw

    out_p = _emb_kernel(b, s, d, per_w)(token_ids.astype(jnp.int32), embdM)
    return out_p[:, :s, :d]
